# baseline (speedup 1.0000x reference)
import functools

import jax
import jax.numpy as jnp
from jax import lax
from jax.experimental import pallas as pl
from jax.experimental.pallas import tpu as pltpu
from jax.experimental.pallas import tpu_sc as plsc

VOCAB = 1000000
EMBED_DIM = 32
BATCH = 16384
HIST = 50

ROW_I32 = EMBED_DIM // 2
NTOT = BATCH * HIST
NC = 2
NS = 16
NW = NC * NS
PER_W = NTOT // NW
CHUNK = 128
NCHUNK = PER_W // CHUNK
GROUP = 8
ROWS_G = GROUP * CHUNK
NGROUP = NCHUNK // GROUP


def _make_gather():
  mesh = plsc.VectorSubcoreMesh(core_axis_name="c", subcore_axis_name="s")

  @functools.partial(
      pl.kernel,
      mesh=mesh,
      out_type=jax.ShapeDtypeStruct((NTOT, ROW_I32), jnp.int32),
      compiler_params=pltpu.CompilerParams(use_tc_tiling_on_sc=False),
      scratch_types=[
          pltpu.VMEM((NCHUNK, CHUNK), jnp.int32),
          pltpu.VMEM((ROWS_G, ROW_I32), jnp.int32),
          pltpu.SemaphoreType.DMA,
      ],
  )
  def gather_kernel(table_hbm, idx_hbm, out_hbm, idx_v, rows_v, sem):
    wid = lax.axis_index("s") * NC + lax.axis_index("c")
    chunk_base = wid * NCHUNK
    row_base = wid * PER_W

    pltpu.sync_copy(idx_hbm.at[pl.ds(chunk_base, NCHUNK)], idx_v)

    def group_body(g, carry):
      def fire(j):
        c = g * GROUP + j
        return pltpu.async_copy(
            table_hbm.at[idx_v.at[c]],
            rows_v.at[pl.ds(j * CHUNK, CHUNK)],
            sem,
        )

      copies = [fire(j) for j in range(GROUP)]
      for cp in copies:
        cp.wait()
      pltpu.sync_copy(
          rows_v, out_hbm.at[pl.ds(row_base + g * ROWS_G, ROWS_G)]
      )
      return carry

    lax.fori_loop(0, NGROUP, group_body, 0)

  return gather_kernel


_gather = _make_gather()


def kernel(index, param):
  idx2 = index.reshape(NTOT // CHUNK, CHUNK)
  param_i32 = jax.lax.bitcast_convert_type(
      param.reshape(VOCAB, ROW_I32, 2), jnp.int32)
  out_i32 = _gather(param_i32, idx2)
  out = jax.lax.bitcast_convert_type(out_i32, jnp.bfloat16)
  return out.reshape(BATCH, HIST, EMBED_DIM)


# speedup vs baseline: 1.0085x; 1.0085x over previous
import functools

import jax
import jax.numpy as jnp
from jax import lax
from jax.experimental import pallas as pl
from jax.experimental.pallas import tpu as pltpu
from jax.experimental.pallas import tpu_sc as plsc

VOCAB = 1000000
EMBED_DIM = 32
BATCH = 16384
HIST = 50

ROW_I32 = EMBED_DIM // 2
NTOT = BATCH * HIST
NC = 2
NS = 16
NW = NC * NS
PER_W = NTOT // NW
CHUNK = 1024
NCHUNK = PER_W // CHUNK
NBUF = 3


def _make_gather():
  mesh = plsc.VectorSubcoreMesh(core_axis_name="c", subcore_axis_name="s")

  @functools.partial(
      pl.kernel,
      mesh=mesh,
      out_type=jax.ShapeDtypeStruct((NTOT, ROW_I32), jnp.int32),
      compiler_params=pltpu.CompilerParams(use_tc_tiling_on_sc=False),
      scratch_types=[
          pltpu.VMEM((NCHUNK, CHUNK), jnp.int32),
          pltpu.VMEM((NBUF, CHUNK, ROW_I32), jnp.int32),
          pltpu.SemaphoreType.DMA((NBUF,)),
          pltpu.SemaphoreType.DMA((NBUF,)),
      ],
  )
  def gather_kernel(table_hbm, idx_hbm, out_hbm, idx_v, rows_v, gsem, wsem):
    wid = lax.axis_index("s") * NC + lax.axis_index("c")
    chunk_base = wid * NCHUNK
    row_base = wid * PER_W

    pltpu.sync_copy(idx_hbm.at[pl.ds(chunk_base, NCHUNK)], idx_v)

    def fire_gather(c):
      b = c % NBUF
      return pltpu.async_copy(
          table_hbm.at[idx_v.at[c]], rows_v.at[b], gsem.at[b])

    def fire_writeback(c):
      b = c % NBUF
      return pltpu.async_copy(
          rows_v.at[b], out_hbm.at[pl.ds(row_base + c * CHUNK, CHUNK)],
          wsem.at[b])

    gth = {}
    wb = {}
    for c in range(NCHUNK):
      b = c % NBUF
      if c >= NBUF:
        wb[c - NBUF].wait()
      gth[c] = fire_gather(c)
      if c >= 1:
        gth[c - 1].wait()
        wb[c - 1] = fire_writeback(c - 1)
    gth[NCHUNK - 1].wait()
    wb[NCHUNK - 1] = fire_writeback(NCHUNK - 1)
    for c in range(NCHUNK - NBUF, NCHUNK):
      wb[c].wait()

  return gather_kernel


_gather = _make_gather()


def kernel(index, param):
  idx2 = index.reshape(NTOT // CHUNK, CHUNK)
  param_i32 = jax.lax.bitcast_convert_type(
      param.reshape(VOCAB, ROW_I32, 2), jnp.int32)
  out_i32 = _gather(param_i32, idx2)
  out = jax.lax.bitcast_convert_type(out_i32, jnp.bfloat16)
  return out.reshape(BATCH, HIST, EMBED_DIM)


# speedup vs baseline: 1.4053x; 1.3935x over previous
import functools

import jax
import jax.numpy as jnp
from jax import lax
from jax.experimental import pallas as pl
from jax.experimental.pallas import tpu as pltpu
from jax.experimental.pallas import tpu_sc as plsc

VOCAB = 1000000
EMBED_DIM = 32
BATCH = 16384
HIST = 50

NTOT = BATCH * HIST
NC = 2
NS = 16
NW = NC * NS
PER_W = NTOT // NW
CHUNK = 1024
NCHUNK = PER_W // CHUNK
NBUF = 3


def _make_gather():
  mesh = plsc.VectorSubcoreMesh(core_axis_name="c", subcore_axis_name="s")

  @functools.partial(
      pl.kernel,
      mesh=mesh,
      out_type=jax.ShapeDtypeStruct((NTOT, EMBED_DIM), jnp.bfloat16),
      compiler_params=pltpu.CompilerParams(use_tc_tiling_on_sc=False),
      scratch_types=[
          pltpu.VMEM((NCHUNK, CHUNK), jnp.int32),
          pltpu.VMEM((NBUF, CHUNK, EMBED_DIM), jnp.bfloat16),
          pltpu.SemaphoreType.DMA((NBUF,)),
          pltpu.SemaphoreType.DMA((NBUF,)),
      ],
  )
  def gather_kernel(table_hbm, idx_hbm, out_hbm, idx_v, rows_v, gsem, wsem):
    wid = lax.axis_index("s") * NC + lax.axis_index("c")
    chunk_base = wid * NCHUNK
    row_base = wid * PER_W

    pltpu.sync_copy(idx_hbm.at[pl.ds(chunk_base, NCHUNK)], idx_v)

    def fire_gather(c):
      b = c % NBUF
      return pltpu.async_copy(
          table_hbm.at[idx_v.at[c]], rows_v.at[b], gsem.at[b])

    def fire_writeback(c):
      b = c % NBUF
      return pltpu.async_copy(
          rows_v.at[b], out_hbm.at[pl.ds(row_base + c * CHUNK, CHUNK)],
          wsem.at[b])

    gth = {}
    wb = {}
    for c in range(NCHUNK):
      b = c % NBUF
      if c >= NBUF:
        wb[c - NBUF].wait()
      gth[c] = fire_gather(c)
      if c >= 1:
        gth[c - 1].wait()
        wb[c - 1] = fire_writeback(c - 1)
    gth[NCHUNK - 1].wait()
    wb[NCHUNK - 1] = fire_writeback(NCHUNK - 1)
    for c in range(NCHUNK - NBUF, NCHUNK):
      wb[c].wait()

  return gather_kernel


_gather = _make_gather()


def kernel(index, param):
  idx2 = index.reshape(NTOT // CHUNK, CHUNK)
  out = _gather(param, idx2)
  return out.reshape(BATCH, HIST, EMBED_DIM)


# speedup vs baseline: 2.3032x; 1.6389x over previous
import functools

import jax
import jax.numpy as jnp
from jax import lax
from jax.experimental import pallas as pl
from jax.experimental.pallas import tpu as pltpu
from jax.experimental.pallas import tpu_sc as plsc

VOCAB = 1000000
EMBED_DIM = 32
BATCH = 16384
HIST = 50

NC = 2
NS = 16
NW = NC * NS
ROWS_W = BATCH // NW
GROUP = 32
NGROUP = ROWS_W // GROUP
NBUF = 3


def _make_gather():
  mesh = plsc.VectorSubcoreMesh(core_axis_name="c", subcore_axis_name="s")

  @functools.partial(
      pl.kernel,
      mesh=mesh,
      out_type=jax.ShapeDtypeStruct((BATCH, HIST, EMBED_DIM), jnp.bfloat16),
      compiler_params=pltpu.CompilerParams(use_tc_tiling_on_sc=False),
      scratch_types=[
          pltpu.VMEM((ROWS_W, HIST), jnp.int32),
          pltpu.VMEM((NBUF, GROUP, HIST, EMBED_DIM), jnp.bfloat16),
          pltpu.SemaphoreType.DMA((NBUF,)),
          pltpu.SemaphoreType.DMA((NBUF,)),
      ],
  )
  def gather_kernel(table_hbm, idx_hbm, out_hbm, idx_v, rows_v, gsem, wsem):
    wid = lax.axis_index("s") * NC + lax.axis_index("c")
    row_base = wid * ROWS_W

    pltpu.sync_copy(idx_hbm.at[pl.ds(row_base, ROWS_W)], idx_v)

    def fire_gathers(g):
      b = g % NBUF
      return [
          pltpu.async_copy(
              table_hbm.at[idx_v.at[g * GROUP + j]],
              rows_v.at[b, j],
              gsem.at[b],
          )
          for j in range(GROUP)
      ]

    def fire_writeback(g):
      b = g % NBUF
      return pltpu.async_copy(
          rows_v.at[b],
          out_hbm.at[pl.ds(row_base + g * GROUP, GROUP)],
          wsem.at[b],
      )

    gth = {}
    wb = {}
    for g in range(NGROUP):
      if g >= NBUF:
        wb[g - NBUF].wait()
      gth[g] = fire_gathers(g)
      if g >= 1:
        for cp in gth.pop(g - 1):
          cp.wait()
        wb[g - 1] = fire_writeback(g - 1)
    for cp in gth.pop(NGROUP - 1):
      cp.wait()
    wb[NGROUP - 1] = fire_writeback(NGROUP - 1)
    for g in range(NGROUP - NBUF, NGROUP):
      wb[g].wait()

  return gather_kernel


_gather = _make_gather()


def kernel(index, param):
  return _gather(param, index)
